# baseline (speedup 1.0000x reference)
import functools

import jax
import jax.numpy as jnp
from jax import lax
from jax.experimental import pallas as pl
from jax.experimental.pallas import tpu as pltpu
from jax.experimental.pallas import tpu_sc as plsc

_B, _H, _W, _P = 8, 512, 512, 50000
_HW = _H * _W
_N = _B * _P
_NT = 392
_NP = _NT * 128 * _B // 32
_TP = _B * _NT * 128
_GRID = 8
_TPG = _NT // _GRID
_CPG = _TPG * 128


def _prep_body(xa_ref, ya_ref, xb_ref, yb_ref, gt_ref, ia_ref, ib_ref, gp_ref):
    g = pl.program_id(0)
    b = lax.broadcasted_iota(jnp.int32, (8, 128), 0) * _HW
    lane = lax.broadcasted_iota(jnp.int32, (8, 128), 1)
    for j in range(_TPG):
        sl = slice(j * 128, (j + 1) * 128)
        col = lane + (g * _CPG + j * 128)
        valid = col < _P
        ia = ya_ref[:, sl] * _W + xa_ref[:, sl] + b
        ib = yb_ref[:, sl] * _W + xb_ref[:, sl] + b
        rows = slice(j * 8, (j + 1) * 8)
        ia_ref[rows, :] = jnp.where(valid, ia, 0)
        ib_ref[rows, :] = jnp.where(valid, ib, 0)
        gp_ref[rows, :] = jnp.where(valid, gt_ref[:, sl], 0.0)


_prep = pl.pallas_call(
    _prep_body,
    grid=(_GRID,),
    in_specs=[
        pl.BlockSpec((8, _CPG), lambda g: (0, g)),
        pl.BlockSpec((8, _CPG), lambda g: (0, g)),
        pl.BlockSpec((8, _CPG), lambda g: (0, g)),
        pl.BlockSpec((8, _CPG), lambda g: (0, g)),
        pl.BlockSpec((8, _CPG), lambda g: (0, g)),
    ],
    out_specs=[
        pl.BlockSpec((_TPG * 8, 128), lambda g: (g, 0)),
        pl.BlockSpec((_TPG * 8, 128), lambda g: (g, 0)),
        pl.BlockSpec((_TPG * 8, 128), lambda g: (g, 0)),
    ],
    out_shape=[
        jax.ShapeDtypeStruct((_TP // 128, 128), jnp.int32),
        jax.ShapeDtypeStruct((_TP // 128, 128), jnp.int32),
        jax.ShapeDtypeStruct((_TP // 128, 128), jnp.float32),
    ],
)

_sc_mesh = plsc.VectorSubcoreMesh(core_axis_name="c", subcore_axis_name="s")


@functools.partial(
    pl.kernel,
    mesh=_sc_mesh,
    out_type=[
        jax.ShapeDtypeStruct((_TP,), jnp.float32),
        jax.ShapeDtypeStruct((_TP,), jnp.float32),
    ],
    scratch_types=[
        pltpu.VMEM((_NP,), jnp.int32),
        pltpu.VMEM((_NP,), jnp.int32),
        pltpu.VMEM((_NP,), jnp.float32),
        pltpu.VMEM((_NP,), jnp.float32),
        pltpu.SemaphoreType.DMA,
    ],
)
def _sc_gather(table, ia, ib, za, zb, iva, ivb, zva, zvb, sem):
    c = lax.axis_index("c")
    s = lax.axis_index("s")
    w = s * 2 + c
    base = pl.multiple_of(w * _NP, _NP)

    pltpu.sync_copy(ia.at[pl.ds(base, _NP)], iva)
    ca = pltpu.async_copy(table.at[iva], zva, sem)
    pltpu.sync_copy(ib.at[pl.ds(base, _NP)], ivb)
    cb = pltpu.async_copy(table.at[ivb], zvb, sem)
    ca.wait()
    cb.wait()
    pltpu.sync_copy(zva, za.at[pl.ds(base, _NP)])
    pltpu.sync_copy(zvb, zb.at[pl.ds(base, _NP)])


def _tc_loss_body(za_ref, zb_ref, gt_ref, out_ref):
    d = za_ref[...] - zb_ref[...]
    g = gt_ref[...]
    m = jnp.abs(g)
    loss = m * jnp.log(1.0 + jnp.exp(-g * d)) + (1.0 - m) * (d * d)
    out_ref[0, 0] = jnp.sum(loss) * (1.0 / _N)


_tc_loss = pl.pallas_call(
    _tc_loss_body,
    out_shape=jax.ShapeDtypeStruct((1, 1), jnp.float32),
    out_specs=pl.BlockSpec(memory_space=pltpu.SMEM),
)


def kernel(input, x_A, y_A, x_B, y_B, ground_truth):
    table = input.reshape(-1)
    ia, ib, gp = _prep(
        x_A.astype(jnp.int32),
        y_A.astype(jnp.int32),
        x_B.astype(jnp.int32),
        y_B.astype(jnp.int32),
        ground_truth,
    )
    za, zb = _sc_gather(table, ia.reshape(-1), ib.reshape(-1))
    shape2d = (_TP // 128, 128)
    out = _tc_loss(za.reshape(shape2d), zb.reshape(shape2d), gp)
    return out[0, 0]


# speedup vs baseline: 1.1419x; 1.1419x over previous
import functools

import jax
import jax.numpy as jnp
from jax import lax
from jax.experimental import pallas as pl
from jax.experimental.pallas import tpu as pltpu
from jax.experimental.pallas import tpu_sc as plsc

_B, _H, _W, _P = 8, 512, 512, 50000
_HW = _H * _W
_N = _B * _P
_NT = 392
_NP = _NT * 128 * _B // 32
_TP = _B * _NT * 128
_GRID = 7
_TPG = _NT // _GRID
_CPG = _TPG * 128


def _prep_body(xa_ref, ya_ref, xb_ref, yb_ref, gt_ref, ia_ref, ib_ref, gp_ref):
    g = pl.program_id(0)
    b = lax.broadcasted_iota(jnp.int32, (8, 128), 0) * _HW
    lane = lax.broadcasted_iota(jnp.int32, (8, 128), 1)
    for j in range(_TPG):
        sl = slice(j * 128, (j + 1) * 128)
        col = lane + (g * _CPG + j * 128)
        valid = col < _P
        ia = ya_ref[:, sl] * _W + xa_ref[:, sl] + b
        ib = yb_ref[:, sl] * _W + xb_ref[:, sl] + b
        ia_ref[:, j, :] = jnp.where(valid, ia, 0)
        ib_ref[:, j, :] = jnp.where(valid, ib, 0)
        gp_ref[:, j, :] = jnp.where(valid, gt_ref[:, sl], 0.0)


_prep = pl.pallas_call(
    _prep_body,
    grid=(_GRID,),
    in_specs=[
        pl.BlockSpec((8, _CPG), lambda g: (0, g)),
        pl.BlockSpec((8, _CPG), lambda g: (0, g)),
        pl.BlockSpec((8, _CPG), lambda g: (0, g)),
        pl.BlockSpec((8, _CPG), lambda g: (0, g)),
        pl.BlockSpec((8, _CPG), lambda g: (0, g)),
    ],
    out_specs=[
        pl.BlockSpec((8, _TPG, 128), lambda g: (0, g, 0)),
        pl.BlockSpec((8, _TPG, 128), lambda g: (0, g, 0)),
        pl.BlockSpec((8, _TPG, 128), lambda g: (0, g, 0)),
    ],
    out_shape=[
        jax.ShapeDtypeStruct((_B, _NT, 128), jnp.int32),
        jax.ShapeDtypeStruct((_B, _NT, 128), jnp.int32),
        jax.ShapeDtypeStruct((_B, _NT, 128), jnp.float32),
    ],
)

_sc_mesh = plsc.VectorSubcoreMesh(core_axis_name="c", subcore_axis_name="s")


@functools.partial(
    pl.kernel,
    mesh=_sc_mesh,
    out_type=[
        jax.ShapeDtypeStruct((_TP,), jnp.float32),
        jax.ShapeDtypeStruct((_TP,), jnp.float32),
    ],
    scratch_types=[
        pltpu.VMEM((_NP,), jnp.int32),
        pltpu.VMEM((_NP,), jnp.int32),
        pltpu.VMEM((_NP,), jnp.float32),
        pltpu.VMEM((_NP,), jnp.float32),
        pltpu.SemaphoreType.DMA,
    ],
)
def _sc_gather(table, ia, ib, za, zb, iva, ivb, zva, zvb, sem):
    c = lax.axis_index("c")
    s = lax.axis_index("s")
    w = s * 2 + c
    base = pl.multiple_of(w * _NP, _NP)

    pltpu.sync_copy(ia.at[pl.ds(base, _NP)], iva)
    ca = pltpu.async_copy(table.at[iva], zva, sem)
    pltpu.sync_copy(ib.at[pl.ds(base, _NP)], ivb)
    cb = pltpu.async_copy(table.at[ivb], zvb, sem)
    ca.wait()
    cb.wait()
    pltpu.sync_copy(zva, za.at[pl.ds(base, _NP)])
    pltpu.sync_copy(zvb, zb.at[pl.ds(base, _NP)])


def _tc_loss_body(za_ref, zb_ref, gt_ref, out_ref):
    d = za_ref[...] - zb_ref[...]
    g = gt_ref[...]
    m = jnp.abs(g)
    loss = m * jnp.log(1.0 + jnp.exp(-g * d)) + (1.0 - m) * (d * d)
    out_ref[0, 0] = jnp.sum(loss) * (1.0 / _N)


_tc_loss = pl.pallas_call(
    _tc_loss_body,
    out_shape=jax.ShapeDtypeStruct((1, 1), jnp.float32),
    out_specs=pl.BlockSpec(memory_space=pltpu.SMEM),
)


def kernel(input, x_A, y_A, x_B, y_B, ground_truth):
    table = input.reshape(-1)
    ia, ib, gp = _prep(
        x_A.astype(jnp.int32),
        y_A.astype(jnp.int32),
        x_B.astype(jnp.int32),
        y_B.astype(jnp.int32),
        ground_truth,
    )
    za, zb = _sc_gather(table, ia.reshape(-1), ib.reshape(-1))
    shape2d = (_TP // 128, 128)
    out = _tc_loss(za.reshape(shape2d), zb.reshape(shape2d), gp.reshape(shape2d))
    return out[0, 0]


# speedup vs baseline: 1.1530x; 1.0097x over previous
import functools

import jax
import jax.numpy as jnp
from jax import lax
from jax.experimental import pallas as pl
from jax.experimental.pallas import tpu as pltpu
from jax.experimental.pallas import tpu_sc as plsc

_B, _H, _W, _P = 8, 512, 512, 50000
_HW = _H * _W
_N = _B * _P
_NT = 392
_NP = _NT * 128 * _B // 32
_TP = _B * _NT * 128
_GRID = 7
_TPG = _NT // _GRID
_CPG = _TPG * 128


def _prep_body(xa_ref, ya_ref, xb_ref, yb_ref, gt_ref, ia_ref, ib_ref, gp_ref):
    g = pl.program_id(0)
    b = lax.broadcasted_iota(jnp.int32, (8, 128), 0) * _HW
    lane = lax.broadcasted_iota(jnp.int32, (8, 128), 1)

    def tile(j, mask):
        sl = slice(j * 128, (j + 1) * 128)
        ia = ya_ref[:, sl] * _W + xa_ref[:, sl] + b
        ib = yb_ref[:, sl] * _W + xb_ref[:, sl] + b
        if mask:
            valid = lane + (g * _CPG + j * 128) < _P
            ia_ref[:, j, :] = jnp.where(valid, ia, 0)
            ib_ref[:, j, :] = jnp.where(valid, ib, 0)
            gp_ref[:, j, :] = jnp.where(valid, gt_ref[:, sl], 0.0)
        else:
            ia_ref[:, j, :] = ia
            ib_ref[:, j, :] = ib
            gp_ref[:, j, :] = gt_ref[:, sl]

    @pl.when(g < _GRID - 1)
    def _():
        for j in range(_TPG):
            tile(j, False)

    @pl.when(g == _GRID - 1)
    def _():
        for j in range(_TPG):
            tile(j, True)


_prep = pl.pallas_call(
    _prep_body,
    grid=(_GRID,),
    in_specs=[
        pl.BlockSpec((8, _CPG), lambda g: (0, g)),
        pl.BlockSpec((8, _CPG), lambda g: (0, g)),
        pl.BlockSpec((8, _CPG), lambda g: (0, g)),
        pl.BlockSpec((8, _CPG), lambda g: (0, g)),
        pl.BlockSpec((8, _CPG), lambda g: (0, g)),
    ],
    out_specs=[
        pl.BlockSpec((8, _TPG, 128), lambda g: (0, g, 0)),
        pl.BlockSpec((8, _TPG, 128), lambda g: (0, g, 0)),
        pl.BlockSpec((8, _TPG, 128), lambda g: (0, g, 0)),
    ],
    out_shape=[
        jax.ShapeDtypeStruct((_B, _NT, 128), jnp.int32),
        jax.ShapeDtypeStruct((_B, _NT, 128), jnp.int32),
        jax.ShapeDtypeStruct((_B, _NT, 128), jnp.float32),
    ],
)

_sc_mesh = plsc.VectorSubcoreMesh(core_axis_name="c", subcore_axis_name="s")


@functools.partial(
    pl.kernel,
    mesh=_sc_mesh,
    out_type=[
        jax.ShapeDtypeStruct((_TP,), jnp.float32),
        jax.ShapeDtypeStruct((_TP,), jnp.float32),
    ],
    scratch_types=[
        pltpu.VMEM((_NP,), jnp.int32),
        pltpu.VMEM((_NP,), jnp.int32),
        pltpu.VMEM((_NP,), jnp.float32),
        pltpu.VMEM((_NP,), jnp.float32),
        pltpu.SemaphoreType.DMA,
    ],
)
def _sc_gather(table, ia, ib, za, zb, iva, ivb, zva, zvb, sem):
    c = lax.axis_index("c")
    s = lax.axis_index("s")
    w = c * 16 + s
    base = pl.multiple_of(w * _NP, _NP)

    pltpu.sync_copy(ia.at[pl.ds(base, _NP)], iva)
    ca = pltpu.async_copy(table.at[iva], zva, sem)
    pltpu.sync_copy(ib.at[pl.ds(base, _NP)], ivb)
    cb = pltpu.async_copy(table.at[ivb], zvb, sem)
    ca.wait()
    cb.wait()
    pltpu.sync_copy(zva, za.at[pl.ds(base, _NP)])
    pltpu.sync_copy(zvb, zb.at[pl.ds(base, _NP)])


def _tc_loss_body(za_ref, zb_ref, gt_ref, out_ref):
    d = za_ref[...] - zb_ref[...]
    g = gt_ref[...]
    m = jnp.abs(g)
    loss = m * jnp.log(1.0 + jnp.exp(-g * d)) + (1.0 - m) * (d * d)
    out_ref[0, 0] = jnp.sum(loss) * (1.0 / _N)


_tc_loss = pl.pallas_call(
    _tc_loss_body,
    out_shape=jax.ShapeDtypeStruct((1, 1), jnp.float32),
    out_specs=pl.BlockSpec(memory_space=pltpu.SMEM),
)


def kernel(input, x_A, y_A, x_B, y_B, ground_truth):
    table = input.reshape(-1)
    ia, ib, gp = _prep(
        x_A.astype(jnp.int32),
        y_A.astype(jnp.int32),
        x_B.astype(jnp.int32),
        y_B.astype(jnp.int32),
        ground_truth,
    )
    za, zb = _sc_gather(table, ia.reshape(-1), ib.reshape(-1))
    shape2d = (_TP // 128, 128)
    out = _tc_loss(za.reshape(shape2d), zb.reshape(shape2d), gp.reshape(shape2d))
    return out[0, 0]


# speedup vs baseline: 1.6151x; 1.4008x over previous
import functools

import jax
import jax.numpy as jnp
from jax import lax
from jax.experimental import pallas as pl
from jax.experimental.pallas import tpu as pltpu
from jax.experimental.pallas import tpu_sc as plsc

_B, _H, _W, _P = 8, 512, 512, 50000
_HW = _H * _W
_N = _B * _P
_NT = 392
_NP = _NT * 128 * _B // 32
_TP = _B * _NT * 128
_GRID = 7
_TPG = _NT // _GRID
_CPG = _TPG * 128


def _prep_body(xa_ref, ya_ref, xb_ref, yb_ref, gt_ref, ia_ref, ib_ref, gp_ref):
    g = pl.program_id(0)
    b = (lax.broadcasted_iota(jnp.int32, (8, 128), 0) & 3) * _HW
    lane = lax.broadcasted_iota(jnp.int32, (8, 128), 1)

    def tile(j, mask):
        sl = slice(j * 128, (j + 1) * 128)
        ia = ya_ref[:, sl] * _W + xa_ref[:, sl] + b
        ib = yb_ref[:, sl] * _W + xb_ref[:, sl] + b
        if mask:
            valid = lane + (g * _CPG + j * 128) < _P
            ia_ref[:, j, :] = jnp.where(valid, ia, 0)
            ib_ref[:, j, :] = jnp.where(valid, ib, 0)
            gp_ref[:, j, :] = jnp.where(valid, gt_ref[:, sl], 0.0)
        else:
            ia_ref[:, j, :] = ia
            ib_ref[:, j, :] = ib
            gp_ref[:, j, :] = gt_ref[:, sl]

    @pl.when(g < _GRID - 1)
    def _():
        for j in range(_TPG):
            tile(j, False)

    @pl.when(g == _GRID - 1)
    def _():
        for j in range(_TPG):
            tile(j, True)


_prep = pl.pallas_call(
    _prep_body,
    grid=(_GRID,),
    in_specs=[
        pl.BlockSpec((8, _CPG), lambda g: (0, g)),
        pl.BlockSpec((8, _CPG), lambda g: (0, g)),
        pl.BlockSpec((8, _CPG), lambda g: (0, g)),
        pl.BlockSpec((8, _CPG), lambda g: (0, g)),
        pl.BlockSpec((8, _CPG), lambda g: (0, g)),
    ],
    out_specs=[
        pl.BlockSpec((8, _TPG, 128), lambda g: (0, g, 0)),
        pl.BlockSpec((8, _TPG, 128), lambda g: (0, g, 0)),
        pl.BlockSpec((8, _TPG, 128), lambda g: (0, g, 0)),
    ],
    out_shape=[
        jax.ShapeDtypeStruct((_B, _NT, 128), jnp.int32),
        jax.ShapeDtypeStruct((_B, _NT, 128), jnp.int32),
        jax.ShapeDtypeStruct((_B, _NT, 128), jnp.float32),
    ],
)

_sc_mesh = plsc.VectorSubcoreMesh(core_axis_name="c", subcore_axis_name="s")


@functools.partial(
    pl.kernel,
    mesh=_sc_mesh,
    out_type=[
        jax.ShapeDtypeStruct((_TP,), jnp.float32),
        jax.ShapeDtypeStruct((_TP,), jnp.float32),
    ],
    scratch_types=[
        pltpu.VMEM((_NP,), jnp.int32),
        pltpu.VMEM((_NP,), jnp.int32),
        pltpu.VMEM((_NP,), jnp.float32),
        pltpu.VMEM((_NP,), jnp.float32),
        pltpu.VMEM_SHARED((4 * _HW,), jnp.float32),
        pltpu.SemaphoreType.DMA,
    ],
)
def _sc_gather(table, ia, ib, za, zb, iva, ivb, zva, zvb, shared, sem):
    c = lax.axis_index("c")
    s = lax.axis_index("s")
    w = c * 16 + s
    base = pl.multiple_of(w * _NP, _NP)

    chunk = 4 * _HW // 16
    pltpu.sync_copy(
        table.at[pl.ds(c * 4 * _HW + s * chunk, chunk)],
        shared.at[pl.ds(s * chunk, chunk)],
    )
    pltpu.sync_copy(ia.at[pl.ds(base, _NP)], iva)
    pltpu.sync_copy(ib.at[pl.ds(base, _NP)], ivb)
    plsc.subcore_barrier()

    ca = pltpu.async_copy(shared.at[iva], zva, sem)
    cb = pltpu.async_copy(shared.at[ivb], zvb, sem)
    ca.wait()
    cb.wait()
    pltpu.sync_copy(zva, za.at[pl.ds(base, _NP)])
    pltpu.sync_copy(zvb, zb.at[pl.ds(base, _NP)])


def _tc_loss_body(za_ref, zb_ref, gt_ref, out_ref):
    d = za_ref[...] - zb_ref[...]
    g = gt_ref[...]
    m = jnp.abs(g)
    loss = m * jnp.log(1.0 + jnp.exp(-g * d)) + (1.0 - m) * (d * d)
    out_ref[0, 0] = jnp.sum(loss) * (1.0 / _N)


_tc_loss = pl.pallas_call(
    _tc_loss_body,
    out_shape=jax.ShapeDtypeStruct((1, 1), jnp.float32),
    out_specs=pl.BlockSpec(memory_space=pltpu.SMEM),
)


def kernel(input, x_A, y_A, x_B, y_B, ground_truth):
    table = input.reshape(-1)
    ia, ib, gp = _prep(
        x_A.astype(jnp.int32),
        y_A.astype(jnp.int32),
        x_B.astype(jnp.int32),
        y_B.astype(jnp.int32),
        ground_truth,
    )
    za, zb = _sc_gather(table, ia.reshape(-1), ib.reshape(-1))
    shape2d = (_TP // 128, 128)
    out = _tc_loss(za.reshape(shape2d), zb.reshape(shape2d), gp.reshape(shape2d))
    return out[0, 0]
